# baseline (speedup 1.0000x reference)
import jax
import jax.numpy as jnp
from jax.experimental import pallas as pl
from jax.experimental.pallas import tpu as pltpu

N = 2048
E = 65536
KTOT = 16 * N
KBLK = 1024
NSTEP = KTOT // KBLK


def _matvec_body(w_ref, f_ref, b_ref, vw_ref, vb_ref, probs_ref, val_ref,
                 acc_ref, vacc_ref):
    k = pl.program_id(0)
    fb = jnp.broadcast_to(f_ref[...], (KBLK, 128))
    part = jnp.dot(w_ref[...], fb, preferred_element_type=jnp.float32)
    vpart = jnp.dot(vw_ref[...], fb, preferred_element_type=jnp.float32)

    @pl.when(k == 0)
    def _():
        acc_ref[...] = part
        vacc_ref[...] = vpart

    @pl.when(k > 0)
    def _():
        acc_ref[...] += part
        vacc_ref[...] += vpart

    @pl.when(k == NSTEP - 1)
    def _():
        logits = acc_ref[:, 0:1] + b_ref[...]
        m = jnp.max(logits)
        e = jnp.exp(logits - m)
        probs_ref[...] = e / jnp.sum(e)
        val_ref[...] = vacc_ref[0:1, 0:1] + vb_ref[...]


def _matvec_softmax(policy_W, f, policy_b, value_W, value_b, interpret=False):
    f2 = f.reshape(KTOT, 1)
    b2 = policy_b.reshape(N, 1)
    vw2 = value_W.reshape(1, KTOT)
    vb2 = value_b.reshape(1, 1)
    probs, val = pl.pallas_call(
        _matvec_body,
        grid=(NSTEP,),
        in_specs=[
            pl.BlockSpec((N, KBLK), lambda k: (0, k)),
            pl.BlockSpec((KBLK, 1), lambda k: (k, 0)),
            pl.BlockSpec((N, 1), lambda k: (0, 0)),
            pl.BlockSpec((1, KBLK), lambda k: (0, k)),
            pl.BlockSpec((1, 1), lambda k: (0, 0)),
        ],
        out_specs=[
            pl.BlockSpec((N, 1), lambda k: (0, 0)),
            pl.BlockSpec((1, 1), lambda k: (0, 0)),
        ],
        out_shape=[
            jax.ShapeDtypeStruct((N, 1), jnp.float32),
            jax.ShapeDtypeStruct((1, 1), jnp.float32),
        ],
        scratch_shapes=[
            pltpu.VMEM((N, 128), jnp.float32),
            pltpu.VMEM((1, 128), jnp.float32),
        ],
        interpret=interpret,
    )(policy_W, f2, b2, vw2, vb2)
    return probs.reshape(N), val.reshape(1)


def _cheb_jnp(x, edge_index, W, b):
    src = edge_index[0]
    dst = edge_index[1]
    ones = jnp.ones((src.shape[0],), dtype=x.dtype)
    deg = jnp.zeros((N,), dtype=x.dtype).at[dst].add(ones)
    dis = jnp.where(deg > 0, 1.0 / jnp.sqrt(jnp.where(deg > 0, deg, 1.0)), 0.0)
    w = -dis[src] * dis[dst]

    def lap(h):
        msg = w[:, None] * jnp.take(h, src, axis=0)
        return jnp.zeros_like(h).at[dst].add(msg)

    Tx0 = x
    Tx1 = lap(Tx0)
    Tx2 = 2.0 * lap(Tx1) - Tx0
    return Tx0 @ W[0] + Tx1 @ W[1] + Tx2 @ W[2] + b


def kernel(subs_x, edge_index, vnr_x, j, cheb_W, cheb_b, vnr_W, vnr_b,
           policy_W, policy_b, value_W, value_b, perm):
    virt_graph = vnr_x @ vnr_W.T + vnr_b
    subs_graph = jax.nn.relu(_cheb_jnp(subs_x, edge_index, cheb_W, cheb_b))
    virt_node = jnp.broadcast_to(virt_graph[j][None, :], (N, 8))
    cat = jnp.concatenate([virt_node, subs_graph], axis=0).reshape(-1)
    fully_conn = jnp.tanh(jnp.take(cat, perm, axis=0))
    return _matvec_softmax(policy_W, fully_conn, policy_b, value_W, value_b)
